# Initial kernel scaffold; baseline (speedup 1.0000x reference)
import jax
import jax.numpy as jnp
from jax.experimental import pallas as pl


def kernel(x, edge_index, deg):
    raise NotImplementedError("write your pallas kernel here")


# speedup vs baseline: 5.4710x; 5.4710x over previous
import functools

import jax
import jax.numpy as jnp
from jax import lax
from jax.experimental import pallas as pl
from jax.experimental.pallas import tpu as pltpu
from jax.experimental.pallas import tpu_sc as plsc

N_NODES = 10000
N_EDGES = 320000
D_FEAT = 128

NC = 2
NS = 16
NW = NC * NS
E_PER_W = N_EDGES // NW
CHUNK = 80
N_CHUNKS = E_PER_W // CHUNK
PAD_NODES = 10240
ROWS_PER_TILE = PAD_NODES // NS


def _sc_partials(x, src, dst, zeros):
  mesh = plsc.VectorSubcoreMesh(core_axis_name="c", subcore_axis_name="s")

  @functools.partial(
      pl.kernel,
      out_type=jax.ShapeDtypeStruct((NC, PAD_NODES, D_FEAT), jnp.float32),
      mesh=mesh,
      scratch_types=[
          pltpu.VMEM((CHUNK,), jnp.int32),
          pltpu.VMEM((CHUNK,), jnp.int32),
          pltpu.VMEM((CHUNK, D_FEAT), jnp.float32),
          pltpu.VMEM_SHARED((PAD_NODES, D_FEAT), jnp.float32),
          pltpu.SemaphoreType.DMA,
      ],
  )
  def k(x_hbm, src_hbm, dst_hbm, zero_hbm, part_hbm, sidx, didx, rows, acc,
        sem):
    cid = lax.axis_index("c")
    sid = lax.axis_index("s")
    wid = sid * NC + cid
    row0 = sid * ROWS_PER_TILE

    pltpu.sync_copy(zero_hbm.at[pl.ds(row0, ROWS_PER_TILE)],
                    acc.at[pl.ds(row0, ROWS_PER_TILE)])
    plsc.subcore_barrier()

    base = wid * E_PER_W

    def body(i, _):
      off = base + i * CHUNK
      pltpu.sync_copy(src_hbm.at[pl.ds(off, CHUNK)], sidx)
      pltpu.sync_copy(dst_hbm.at[pl.ds(off, CHUNK)], didx)
      pltpu.async_copy(x_hbm.at[sidx], rows, sem).wait()
      pltpu.sync_copy(rows, acc.at[didx], add=True)
      return ()

    lax.fori_loop(0, N_CHUNKS, body, ())

    plsc.subcore_barrier()
    pltpu.sync_copy(acc.at[pl.ds(row0, ROWS_PER_TILE)],
                    part_hbm.at[cid, pl.ds(row0, ROWS_PER_TILE)])

  return k(x, src, dst, zeros)


def _combine_kernel(p_ref, deg_ref, o_ref):
  o_ref[...] = (p_ref[0] + p_ref[1]) / deg_ref[...]


def _combine(partials, deg2d):
  rb = 1000
  return pl.pallas_call(
      _combine_kernel,
      out_shape=jax.ShapeDtypeStruct((N_NODES, D_FEAT), jnp.float32),
      grid=(N_NODES // rb,),
      in_specs=[
          pl.BlockSpec((NC, rb, D_FEAT), lambda i: (0, i, 0)),
          pl.BlockSpec((rb, 1), lambda i: (i, 0)),
      ],
      out_specs=pl.BlockSpec((rb, D_FEAT), lambda i: (i, 0)),
  )(partials, deg2d)


@jax.jit
def kernel(x, edge_index, deg):
  src = edge_index[0].astype(jnp.int32)
  dst = edge_index[1].astype(jnp.int32)
  zeros = jnp.zeros((PAD_NODES, D_FEAT), jnp.float32)
  partials = _sc_partials(x, src, dst, zeros)
  return _combine(partials, deg.reshape(N_NODES, 1))


# speedup vs baseline: 12.2819x; 2.2449x over previous
import functools

import jax
import jax.numpy as jnp
from jax import lax
from jax.experimental import pallas as pl
from jax.experimental.pallas import tpu as pltpu
from jax.experimental.pallas import tpu_sc as plsc

N_NODES = 10000
N_EDGES = 320000
D_FEAT = 128

NC = 2
NS = 16
NW = NC * NS
E_PER_W = N_EDGES // NW
CHUNK = 80
N_CHUNKS = E_PER_W // CHUNK
NBUF = 4
GLA = 2
IBUF = 8
ILA = 6
PAD_NODES = 10240
ROWS_PER_TILE = PAD_NODES // NS


def _sc_partials(x, src, dst, zeros):
  mesh = plsc.VectorSubcoreMesh(core_axis_name="c", subcore_axis_name="s")

  @functools.partial(
      pl.kernel,
      out_type=jax.ShapeDtypeStruct((NC, PAD_NODES, D_FEAT), jnp.float32),
      mesh=mesh,
      scratch_types=[
          pltpu.VMEM((IBUF, CHUNK), jnp.int32),
          pltpu.VMEM((IBUF, CHUNK), jnp.int32),
          pltpu.VMEM((NBUF, CHUNK, D_FEAT), jnp.float32),
          pltpu.VMEM_SHARED((PAD_NODES, D_FEAT), jnp.float32),
          pltpu.SemaphoreType.DMA((IBUF,)),
          pltpu.SemaphoreType.DMA((NBUF,)),
          pltpu.SemaphoreType.DMA((NBUF,)),
      ],
  )
  def k(x_hbm, src_hbm, dst_hbm, zero_hbm, part_hbm, sidx, didx, rows, acc,
        isem, gsem, ssem):
    cid = lax.axis_index("c")
    sid = lax.axis_index("s")
    wid = sid * NC + cid
    row0 = sid * ROWS_PER_TILE
    ebase = wid * E_PER_W

    def start_idx(c, slot):
      off = ebase + c * CHUNK
      pltpu.async_copy(src_hbm.at[pl.ds(off, CHUNK)], sidx.at[slot],
                       isem.at[slot])
      pltpu.async_copy(dst_hbm.at[pl.ds(off, CHUNK)], didx.at[slot],
                       isem.at[slot])

    def wait_idx(slot):
      pltpu.make_async_copy(src_hbm.at[pl.ds(0, CHUNK)], sidx.at[slot],
                            isem.at[slot]).wait()
      pltpu.make_async_copy(dst_hbm.at[pl.ds(0, CHUNK)], didx.at[slot],
                            isem.at[slot]).wait()

    def start_gather(slot, b):
      pltpu.async_copy(x_hbm.at[sidx.at[slot]], rows.at[b], gsem.at[b])

    def wait_gather(b):
      pltpu.make_async_copy(x_hbm.at[sidx.at[0]], rows.at[b],
                            gsem.at[b]).wait()

    def start_scatter(slot, b):
      pltpu.async_copy(rows.at[b], acc.at[didx.at[slot]], ssem.at[b],
                       add=True)

    def wait_scatter(b):
      pltpu.make_async_copy(rows.at[b], acc.at[didx.at[0]], ssem.at[b]).wait()

    def step(j):
      b = j % NBUF
      wait_gather(b)
      start_scatter(j % IBUF, b)
      if j >= GLA:
        wait_scatter((j + GLA) % NBUF)
      if j + ILA < N_CHUNKS:
        start_idx(j + ILA, (j + ILA) % IBUF)
      if j + GLA < N_CHUNKS:
        wait_idx((j + GLA) % IBUF)
        start_gather((j + GLA) % IBUF, (j + GLA) % NBUF)

    for c in range(ILA):
      start_idx(c, c)
    for c in range(GLA):
      wait_idx(c)
      start_gather(c, c)
    pltpu.sync_copy(zero_hbm.at[pl.ds(row0, ROWS_PER_TILE)],
                    acc.at[pl.ds(row0, ROWS_PER_TILE)])
    plsc.subcore_barrier()

    head = IBUF
    n_main = (N_CHUNKS - head - (ILA + GLA)) // IBUF
    for j in range(head):
      step(j)

    def body(i, _):
      j0 = head + i * IBUF
      for kk in range(IBUF):
        jj = j0 + kk
        b = kk % NBUF
        wait_gather(b)
        start_scatter(kk, b)
        wait_scatter((kk + GLA) % NBUF)
        start_idx(jj + ILA, (kk + ILA) % IBUF)
        wait_idx((kk + GLA) % IBUF)
        start_gather((kk + GLA) % IBUF, (kk + GLA) % NBUF)
      return ()

    lax.fori_loop(0, n_main, body, ())

    for j in range(head + n_main * IBUF, N_CHUNKS):
      step(j)

    for j in range(N_CHUNKS - GLA, N_CHUNKS):
      wait_scatter(j % NBUF)

    plsc.subcore_barrier()
    pltpu.sync_copy(acc.at[pl.ds(row0, ROWS_PER_TILE)],
                    part_hbm.at[cid, pl.ds(row0, ROWS_PER_TILE)])

  return k(x, src, dst, zeros)


def _combine_kernel(p_ref, deg_ref, o_ref):
  o_ref[...] = (p_ref[0] + p_ref[1]) / deg_ref[...]


def _combine(partials, deg2d):
  rb = 1000
  return pl.pallas_call(
      _combine_kernel,
      out_shape=jax.ShapeDtypeStruct((N_NODES, D_FEAT), jnp.float32),
      grid=(N_NODES // rb,),
      in_specs=[
          pl.BlockSpec((NC, rb, D_FEAT), lambda i: (0, i, 0)),
          pl.BlockSpec((rb, 1), lambda i: (i, 0)),
      ],
      out_specs=pl.BlockSpec((rb, D_FEAT), lambda i: (i, 0)),
  )(partials, deg2d)


@jax.jit
def kernel(x, edge_index, deg):
  src = edge_index[0].astype(jnp.int32)
  dst = edge_index[1].astype(jnp.int32)
  zeros = jnp.zeros((PAD_NODES, D_FEAT), jnp.float32)
  partials = _sc_partials(x, src, dst, zeros)
  return _combine(partials, deg.reshape(N_NODES, 1))


# speedup vs baseline: 12.6385x; 1.0290x over previous
import functools

import jax
import jax.numpy as jnp
from jax import lax
from jax.experimental import pallas as pl
from jax.experimental.pallas import tpu as pltpu
from jax.experimental.pallas import tpu_sc as plsc

N_NODES = 10000
N_EDGES = 320000
D_FEAT = 128

NC = 2
NS = 16
NW = NC * NS
E_PER_W = N_EDGES // NW
CHUNK = 80
N_CHUNKS = E_PER_W // CHUNK
NBUF = 4
GLA = 2
IBUF = 8
ILA = 6
PAD_NODES = 10240
ROWS_PER_TILE = PAD_NODES // NS


def _sc_partials(x, src, dst):
  mesh = plsc.VectorSubcoreMesh(core_axis_name="c", subcore_axis_name="s")

  @functools.partial(
      pl.kernel,
      out_type=jax.ShapeDtypeStruct((NC, PAD_NODES, D_FEAT), jnp.float32),
      mesh=mesh,
      scratch_types=[
          pltpu.VMEM((IBUF, CHUNK), jnp.int32),
          pltpu.VMEM((IBUF, CHUNK), jnp.int32),
          pltpu.VMEM((NBUF, CHUNK, D_FEAT), jnp.float32),
          pltpu.VMEM_SHARED((PAD_NODES, D_FEAT), jnp.float32),
          pltpu.SemaphoreType.DMA((IBUF,)),
          pltpu.SemaphoreType.DMA((NBUF,)),
          pltpu.SemaphoreType.DMA((NBUF,)),
      ],
  )
  def k(x_hbm, src_hbm, dst_hbm, part_hbm, sidx, didx, rows, acc,
        isem, gsem, ssem):
    cid = lax.axis_index("c")
    sid = lax.axis_index("s")
    wid = sid * NC + cid
    row0 = sid * ROWS_PER_TILE
    ebase = wid * E_PER_W

    def start_idx(c, slot):
      off = ebase + c * CHUNK
      pltpu.async_copy(src_hbm.at[pl.ds(off, CHUNK)], sidx.at[slot],
                       isem.at[slot])
      pltpu.async_copy(dst_hbm.at[pl.ds(off, CHUNK)], didx.at[slot],
                       isem.at[slot])

    def wait_idx(slot):
      pltpu.make_async_copy(src_hbm.at[pl.ds(0, CHUNK)], sidx.at[slot],
                            isem.at[slot]).wait()
      pltpu.make_async_copy(dst_hbm.at[pl.ds(0, CHUNK)], didx.at[slot],
                            isem.at[slot]).wait()

    def start_gather(slot, b):
      pltpu.async_copy(x_hbm.at[sidx.at[slot]], rows.at[b], gsem.at[b])

    def wait_gather(b):
      pltpu.make_async_copy(x_hbm.at[sidx.at[0]], rows.at[b],
                            gsem.at[b]).wait()

    def start_scatter(slot, b):
      pltpu.async_copy(rows.at[b], acc.at[didx.at[slot]], ssem.at[b],
                       add=True)

    def wait_scatter(b):
      pltpu.make_async_copy(rows.at[b], acc.at[didx.at[0]], ssem.at[b]).wait()

    def step(j):
      b = j % NBUF
      wait_gather(b)
      start_scatter(j % IBUF, b)
      if j >= GLA:
        wait_scatter((j + GLA) % NBUF)
      if j + ILA < N_CHUNKS:
        start_idx(j + ILA, (j + ILA) % IBUF)
      if j + GLA < N_CHUNKS:
        wait_idx((j + GLA) % IBUF)
        start_gather((j + GLA) % IBUF, (j + GLA) % NBUF)

    for c in range(ILA):
      start_idx(c, c)
    zrow = rows.at[0]

    def zbody(r, _):
      for c8 in range(D_FEAT // 16):
        zrow[r, pl.ds(c8 * 16, 16)] = jnp.zeros((16,), jnp.float32)
      return ()

    lax.fori_loop(0, CHUNK, zbody, ())
    for p in range(ROWS_PER_TILE // CHUNK):
      pltpu.sync_copy(zrow, acc.at[pl.ds(row0 + p * CHUNK, CHUNK)])
    for c in range(GLA):
      wait_idx(c)
      start_gather(c, c)
    plsc.subcore_barrier()

    head = IBUF
    n_main = (N_CHUNKS - head - (ILA + GLA)) // IBUF
    for j in range(head):
      step(j)

    def body(i, _):
      j0 = head + i * IBUF
      for kk in range(IBUF):
        jj = j0 + kk
        b = kk % NBUF
        wait_gather(b)
        start_scatter(kk, b)
        wait_scatter((kk + GLA) % NBUF)
        start_idx(jj + ILA, (kk + ILA) % IBUF)
        wait_idx((kk + GLA) % IBUF)
        start_gather((kk + GLA) % IBUF, (kk + GLA) % NBUF)
      return ()

    lax.fori_loop(0, n_main, body, ())

    for j in range(head + n_main * IBUF, N_CHUNKS):
      step(j)

    for j in range(N_CHUNKS - GLA, N_CHUNKS):
      wait_scatter(j % NBUF)

    plsc.subcore_barrier()
    pltpu.sync_copy(acc.at[pl.ds(row0, ROWS_PER_TILE)],
                    part_hbm.at[cid, pl.ds(row0, ROWS_PER_TILE)])

  return k(x, src, dst)


def _combine_kernel(p_ref, deg_ref, o_ref):
  o_ref[...] = (p_ref[0] + p_ref[1]) / deg_ref[...]


def _combine(partials, deg2d):
  rb = 1000
  return pl.pallas_call(
      _combine_kernel,
      out_shape=jax.ShapeDtypeStruct((N_NODES, D_FEAT), jnp.float32),
      grid=(N_NODES // rb,),
      in_specs=[
          pl.BlockSpec((NC, rb, D_FEAT), lambda i: (0, i, 0)),
          pl.BlockSpec((rb, 1), lambda i: (i, 0)),
      ],
      out_specs=pl.BlockSpec((rb, D_FEAT), lambda i: (i, 0)),
  )(partials, deg2d)


@jax.jit
def kernel(x, edge_index, deg):
  src = edge_index[0].astype(jnp.int32)
  dst = edge_index[1].astype(jnp.int32)
  partials = _sc_partials(x, src, dst)
  return _combine(partials, deg.reshape(N_NODES, 1))


# speedup vs baseline: 12.6775x; 1.0031x over previous
import functools

import jax
import jax.numpy as jnp
from jax import lax
from jax.experimental import pallas as pl
from jax.experimental.pallas import tpu as pltpu
from jax.experimental.pallas import tpu_sc as plsc

N_NODES = 10000
N_EDGES = 320000
D_FEAT = 128

NC = 2
NS = 16
NW = NC * NS
E_PER_W = N_EDGES // NW
CHUNK = 80
N_CHUNKS = E_PER_W // CHUNK
NBUF = 4
GLA = 2
IBUF = 8
ILA = 6
PAD_NODES = 10240
ROWS_PER_TILE = PAD_NODES // NS


def _sc_partials(x, src, dst):
  mesh = plsc.VectorSubcoreMesh(core_axis_name="c", subcore_axis_name="s")

  @functools.partial(
      pl.kernel,
      out_type=jax.ShapeDtypeStruct((NC, PAD_NODES, D_FEAT), jnp.float32),
      mesh=mesh,
      compiler_params=pltpu.CompilerParams(skip_device_barrier=True),
      scratch_types=[
          pltpu.VMEM((IBUF, CHUNK), jnp.int32),
          pltpu.VMEM((IBUF, CHUNK), jnp.int32),
          pltpu.VMEM((NBUF, CHUNK, D_FEAT), jnp.float32),
          pltpu.VMEM_SHARED((PAD_NODES, D_FEAT), jnp.float32),
          pltpu.SemaphoreType.DMA((IBUF,)),
          pltpu.SemaphoreType.DMA((NBUF,)),
          pltpu.SemaphoreType.DMA((NBUF,)),
      ],
  )
  def k(x_hbm, src_hbm, dst_hbm, part_hbm, sidx, didx, rows, acc,
        isem, gsem, ssem):
    cid = lax.axis_index("c")
    sid = lax.axis_index("s")
    wid = sid * NC + cid
    row0 = sid * ROWS_PER_TILE
    ebase = wid * E_PER_W

    def start_idx(c, slot):
      off = ebase + c * CHUNK
      pltpu.async_copy(src_hbm.at[pl.ds(off, CHUNK)], sidx.at[slot],
                       isem.at[slot])
      pltpu.async_copy(dst_hbm.at[pl.ds(off, CHUNK)], didx.at[slot],
                       isem.at[slot])

    def wait_idx(slot):
      pltpu.make_async_copy(src_hbm.at[pl.ds(0, CHUNK)], sidx.at[slot],
                            isem.at[slot]).wait()
      pltpu.make_async_copy(dst_hbm.at[pl.ds(0, CHUNK)], didx.at[slot],
                            isem.at[slot]).wait()

    def start_gather(slot, b):
      pltpu.async_copy(x_hbm.at[sidx.at[slot]], rows.at[b], gsem.at[b])

    def wait_gather(b):
      pltpu.make_async_copy(x_hbm.at[sidx.at[0]], rows.at[b],
                            gsem.at[b]).wait()

    def start_scatter(slot, b):
      pltpu.async_copy(rows.at[b], acc.at[didx.at[slot]], ssem.at[b],
                       add=True)

    def wait_scatter(b):
      pltpu.make_async_copy(rows.at[b], acc.at[didx.at[0]], ssem.at[b]).wait()

    def step(j):
      b = j % NBUF
      wait_gather(b)
      start_scatter(j % IBUF, b)
      if j >= GLA:
        wait_scatter((j + GLA) % NBUF)
      if j + ILA < N_CHUNKS:
        start_idx(j + ILA, (j + ILA) % IBUF)
      if j + GLA < N_CHUNKS:
        wait_idx((j + GLA) % IBUF)
        start_gather((j + GLA) % IBUF, (j + GLA) % NBUF)

    for c in range(ILA):
      start_idx(c, c)
    zrow = rows.at[0]

    def zbody(r, _):
      for c8 in range(D_FEAT // 16):
        zrow[r, pl.ds(c8 * 16, 16)] = jnp.zeros((16,), jnp.float32)
      return ()

    lax.fori_loop(0, CHUNK, zbody, ())
    for p in range(ROWS_PER_TILE // CHUNK):
      pltpu.sync_copy(zrow, acc.at[pl.ds(row0 + p * CHUNK, CHUNK)])
    for c in range(GLA):
      wait_idx(c)
      start_gather(c, c)
    plsc.subcore_barrier()

    head = IBUF
    n_main = (N_CHUNKS - head - (ILA + GLA)) // IBUF
    for j in range(head):
      step(j)

    def body(i, _):
      j0 = head + i * IBUF
      for kk in range(IBUF):
        jj = j0 + kk
        b = kk % NBUF
        wait_gather(b)
        start_scatter(kk, b)
        wait_scatter((kk + GLA) % NBUF)
        start_idx(jj + ILA, (kk + ILA) % IBUF)
        wait_idx((kk + GLA) % IBUF)
        start_gather((kk + GLA) % IBUF, (kk + GLA) % NBUF)
      return ()

    lax.fori_loop(0, n_main, body, ())

    for j in range(head + n_main * IBUF, N_CHUNKS):
      step(j)

    for j in range(N_CHUNKS - GLA, N_CHUNKS):
      wait_scatter(j % NBUF)

    plsc.subcore_barrier()
    pltpu.sync_copy(acc.at[pl.ds(row0, ROWS_PER_TILE)],
                    part_hbm.at[cid, pl.ds(row0, ROWS_PER_TILE)])

  return k(x, src, dst)


def _combine_kernel(p_ref, deg_ref, o_ref):
  o_ref[...] = (p_ref[0] + p_ref[1]) / deg_ref[...]


def _combine(partials, deg2d):
  rb = 1000
  return pl.pallas_call(
      _combine_kernel,
      out_shape=jax.ShapeDtypeStruct((N_NODES, D_FEAT), jnp.float32),
      grid=(N_NODES // rb,),
      in_specs=[
          pl.BlockSpec((NC, rb, D_FEAT), lambda i: (0, i, 0)),
          pl.BlockSpec((rb, 1), lambda i: (i, 0)),
      ],
      out_specs=pl.BlockSpec((rb, D_FEAT), lambda i: (i, 0)),
  )(partials, deg2d)


@jax.jit
def kernel(x, edge_index, deg):
  src = edge_index[0].astype(jnp.int32)
  dst = edge_index[1].astype(jnp.int32)
  partials = _sc_partials(x, src, dst)
  return _combine(partials, deg.reshape(N_NODES, 1))


# speedup vs baseline: 13.7375x; 1.0836x over previous
import functools

import jax
import jax.numpy as jnp
from jax import lax
from jax.experimental import pallas as pl
from jax.experimental.pallas import tpu as pltpu
from jax.experimental.pallas import tpu_sc as plsc

N_NODES = 10000
N_EDGES = 320000
D_FEAT = 128

NC = 2
NS = 16
NW = NC * NS
E_PER_W = N_EDGES // NW
CHUNK = 80
N_CHUNKS = E_PER_W // CHUNK
NBUF = 4
GLA = 2
IBUF = 8
ILA = 6
assert ILA + GLA <= IBUF and 2 * GLA <= NBUF and GLA < ILA
PAD_NODES = 10240
ROWS_PER_TILE = PAD_NODES // NS


def _sc_partials(x, edge_index):
  mesh = plsc.VectorSubcoreMesh(core_axis_name="c", subcore_axis_name="s")

  @functools.partial(
      pl.kernel,
      out_type=jax.ShapeDtypeStruct((NC, PAD_NODES, D_FEAT), jnp.float32),
      mesh=mesh,
      scratch_types=[
          pltpu.VMEM((IBUF, CHUNK), jnp.int32),
          pltpu.VMEM((IBUF, CHUNK), jnp.int32),
          pltpu.VMEM((NBUF, CHUNK, D_FEAT), jnp.float32),
          pltpu.VMEM_SHARED((PAD_NODES, D_FEAT), jnp.float32),
          pltpu.SemaphoreType.DMA((IBUF,)),
          pltpu.SemaphoreType.DMA((NBUF,)),
          pltpu.SemaphoreType.DMA((NBUF,)),
      ],
  )
  def k(x_hbm, e_hbm, part_hbm, sidx, didx, rows, acc,
        isem, gsem, ssem):
    cid = lax.axis_index("c")
    sid = lax.axis_index("s")
    wid = sid * NC + cid
    row0 = sid * ROWS_PER_TILE
    ebase = wid * E_PER_W

    def start_idx(c, slot):
      off = ebase + c * CHUNK
      pltpu.async_copy(e_hbm.at[pl.ds(off, CHUNK)], sidx.at[slot],
                       isem.at[slot])
      pltpu.async_copy(e_hbm.at[pl.ds(N_EDGES + off, CHUNK)], didx.at[slot],
                       isem.at[slot])

    def wait_idx(slot):
      pltpu.make_async_copy(e_hbm.at[pl.ds(0, CHUNK)], sidx.at[slot],
                            isem.at[slot]).wait()
      pltpu.make_async_copy(e_hbm.at[pl.ds(0, CHUNK)], didx.at[slot],
                            isem.at[slot]).wait()

    def start_gather(slot, b):
      pltpu.async_copy(x_hbm.at[sidx.at[slot]], rows.at[b], gsem.at[b])

    def wait_gather(b):
      pltpu.make_async_copy(x_hbm.at[sidx.at[0]], rows.at[b],
                            gsem.at[b]).wait()

    def start_scatter(slot, b):
      pltpu.async_copy(rows.at[b], acc.at[didx.at[slot]], ssem.at[b],
                       add=True)

    def wait_scatter(b):
      pltpu.make_async_copy(rows.at[b], acc.at[didx.at[0]], ssem.at[b]).wait()

    def step(j):
      b = j % NBUF
      wait_gather(b)
      start_scatter(j % IBUF, b)
      if j >= GLA:
        wait_scatter((j + GLA) % NBUF)
      if j + ILA < N_CHUNKS:
        start_idx(j + ILA, (j + ILA) % IBUF)
      if j + GLA < N_CHUNKS:
        wait_idx((j + GLA) % IBUF)
        start_gather((j + GLA) % IBUF, (j + GLA) % NBUF)

    for c in range(ILA):
      start_idx(c, c)
    zrow = rows.at[0]

    def zbody(r, _):
      for c8 in range(D_FEAT // 16):
        zrow[r, pl.ds(c8 * 16, 16)] = jnp.zeros((16,), jnp.float32)
      return ()

    lax.fori_loop(0, CHUNK, zbody, ())
    for p in range(ROWS_PER_TILE // CHUNK):
      pltpu.sync_copy(zrow, acc.at[pl.ds(row0 + p * CHUNK, CHUNK)])
    for c in range(GLA):
      wait_idx(c)
      start_gather(c, c)
    plsc.subcore_barrier()

    head = IBUF
    n_main = (N_CHUNKS - head - (ILA + GLA)) // IBUF
    for j in range(head):
      step(j)

    def body(i, _):
      j0 = head + i * IBUF
      for kk in range(IBUF):
        jj = j0 + kk
        b = kk % NBUF
        wait_gather(b)
        start_scatter(kk, b)
        wait_scatter((kk + GLA) % NBUF)
        start_idx(jj + ILA, (kk + ILA) % IBUF)
        wait_idx((kk + GLA) % IBUF)
        start_gather((kk + GLA) % IBUF, (kk + GLA) % NBUF)
      return ()

    lax.fori_loop(0, n_main, body, ())

    for j in range(head + n_main * IBUF, N_CHUNKS):
      step(j)

    for j in range(N_CHUNKS - GLA, N_CHUNKS):
      wait_scatter(j % NBUF)

    plsc.subcore_barrier()
    pltpu.sync_copy(acc.at[pl.ds(row0, ROWS_PER_TILE)],
                    part_hbm.at[cid, pl.ds(row0, ROWS_PER_TILE)])

  return k(x, edge_index)


def _combine_kernel(p_ref, deg_ref, o_ref):
  o_ref[...] = (p_ref[0] + p_ref[1]) / deg_ref[...]


def _combine(partials, deg2d):
  rb = 2000
  return pl.pallas_call(
      _combine_kernel,
      out_shape=jax.ShapeDtypeStruct((N_NODES, D_FEAT), jnp.float32),
      grid=(N_NODES // rb,),
      in_specs=[
          pl.BlockSpec((NC, rb, D_FEAT), lambda i: (0, i, 0)),
          pl.BlockSpec((rb, 1), lambda i: (i, 0)),
      ],
      out_specs=pl.BlockSpec((rb, D_FEAT), lambda i: (i, 0)),
  )(partials, deg2d)


@jax.jit
def kernel(x, edge_index, deg):
  partials = _sc_partials(x, edge_index.astype(jnp.int32).reshape(2 * N_EDGES))
  return _combine(partials, deg.reshape(N_NODES, 1))


# speedup vs baseline: 13.8357x; 1.0071x over previous
import functools

import jax
import jax.numpy as jnp
from jax import lax
from jax.experimental import pallas as pl
from jax.experimental.pallas import tpu as pltpu
from jax.experimental.pallas import tpu_sc as plsc

N_NODES = 10000
N_EDGES = 320000
D_FEAT = 128

NC = 2
NS = 16
NW = NC * NS
E_PER_W = N_EDGES // NW
CHUNK = 80
N_CHUNKS = E_PER_W // CHUNK
NBUF = 4
GLA = 2
IBUF = 8
ILA = 6
assert ILA + GLA <= IBUF and 2 * GLA <= NBUF and GLA < ILA
PAD_NODES = 10240
ROWS_PER_TILE = PAD_NODES // NS


def _sc_partials(x, edge_index):
  mesh = plsc.VectorSubcoreMesh(core_axis_name="c", subcore_axis_name="s")

  @functools.partial(
      pl.kernel,
      out_type=jax.ShapeDtypeStruct((NC, PAD_NODES, D_FEAT), jnp.float32),
      mesh=mesh,
      scratch_types=[
          pltpu.VMEM((IBUF, CHUNK), jnp.int32),
          pltpu.VMEM((IBUF, CHUNK), jnp.int32),
          pltpu.VMEM((NBUF, CHUNK, D_FEAT), jnp.float32),
          pltpu.VMEM_SHARED((PAD_NODES, D_FEAT), jnp.float32),
          pltpu.SemaphoreType.DMA((IBUF,)),
          pltpu.SemaphoreType.DMA((NBUF,)),
          pltpu.SemaphoreType.DMA((NBUF,)),
      ],
  )
  def k(x_hbm, e_hbm, part_hbm, sidx, didx, rows, acc,
        isem, gsem, ssem):
    cid = lax.axis_index("c")
    sid = lax.axis_index("s")
    wid = sid * NC + cid
    row0 = sid * ROWS_PER_TILE
    ebase = wid * E_PER_W

    def start_idx(c, slot):
      off = ebase + c * CHUNK
      pltpu.async_copy(e_hbm.at[pl.ds(off, CHUNK)], sidx.at[slot],
                       isem.at[slot])
      pltpu.async_copy(e_hbm.at[pl.ds(N_EDGES + off, CHUNK)], didx.at[slot],
                       isem.at[slot])

    def wait_idx(slot):
      pltpu.make_async_copy(e_hbm.at[pl.ds(0, CHUNK)], sidx.at[slot],
                            isem.at[slot]).wait()
      pltpu.make_async_copy(e_hbm.at[pl.ds(0, CHUNK)], didx.at[slot],
                            isem.at[slot]).wait()

    def start_gather(slot, b):
      pltpu.async_copy(x_hbm.at[sidx.at[slot]], rows.at[b], gsem.at[b])

    def wait_gather(b):
      pltpu.make_async_copy(x_hbm.at[sidx.at[0]], rows.at[b],
                            gsem.at[b]).wait()

    def start_scatter(slot, b):
      pltpu.async_copy(rows.at[b], acc.at[didx.at[slot]], ssem.at[b],
                       add=True)

    def wait_scatter(b):
      pltpu.make_async_copy(rows.at[b], acc.at[didx.at[0]], ssem.at[b]).wait()

    def step(j):
      b = j % NBUF
      wait_gather(b)
      start_scatter(j % IBUF, b)
      if j >= GLA:
        wait_scatter((j + GLA) % NBUF)
      if j + ILA < N_CHUNKS:
        start_idx(j + ILA, (j + ILA) % IBUF)
      if j + GLA < N_CHUNKS:
        wait_idx((j + GLA) % IBUF)
        start_gather((j + GLA) % IBUF, (j + GLA) % NBUF)

    for c in range(ILA):
      start_idx(c, c)
    for c in range(GLA):
      wait_idx(c)
      start_gather(c, c)
    zrow = rows.at[GLA]

    def zbody(r, _):
      for c8 in range(D_FEAT // 16):
        zrow[r, pl.ds(c8 * 16, 16)] = jnp.zeros((16,), jnp.float32)
      return ()

    lax.fori_loop(0, CHUNK, zbody, ())
    for p in range(ROWS_PER_TILE // CHUNK):
      pltpu.sync_copy(zrow, acc.at[pl.ds(row0 + p * CHUNK, CHUNK)])
    plsc.subcore_barrier()

    head = IBUF
    n_main = (N_CHUNKS - head - (ILA + GLA)) // IBUF
    for j in range(head):
      step(j)

    def body(i, _):
      j0 = head + i * IBUF
      for kk in range(IBUF):
        jj = j0 + kk
        b = kk % NBUF
        wait_gather(b)
        start_scatter(kk, b)
        wait_scatter((kk + GLA) % NBUF)
        start_idx(jj + ILA, (kk + ILA) % IBUF)
        wait_idx((kk + GLA) % IBUF)
        start_gather((kk + GLA) % IBUF, (kk + GLA) % NBUF)
      return ()

    lax.fori_loop(0, n_main, body, ())

    for j in range(head + n_main * IBUF, N_CHUNKS):
      step(j)

    for j in range(N_CHUNKS - GLA, N_CHUNKS):
      wait_scatter(j % NBUF)

    plsc.subcore_barrier()
    pltpu.sync_copy(acc.at[pl.ds(row0, ROWS_PER_TILE)],
                    part_hbm.at[cid, pl.ds(row0, ROWS_PER_TILE)])

  return k(x, edge_index)


def _combine_kernel(p_ref, deg_ref, o_ref):
  o_ref[...] = (p_ref[0] + p_ref[1]) / deg_ref[...]


def _combine(partials, deg2d):
  rb = 2000
  return pl.pallas_call(
      _combine_kernel,
      out_shape=jax.ShapeDtypeStruct((N_NODES, D_FEAT), jnp.float32),
      grid=(N_NODES // rb,),
      in_specs=[
          pl.BlockSpec((NC, rb, D_FEAT), lambda i: (0, i, 0)),
          pl.BlockSpec((rb, 1), lambda i: (i, 0)),
      ],
      out_specs=pl.BlockSpec((rb, D_FEAT), lambda i: (i, 0)),
  )(partials, deg2d)


@jax.jit
def kernel(x, edge_index, deg):
  partials = _sc_partials(x, edge_index.astype(jnp.int32).reshape(2 * N_EDGES))
  return _combine(partials, deg.reshape(N_NODES, 1))
